# baseline (speedup 1.0000x reference)
import functools

import jax
import jax.numpy as jnp
from jax import lax
from jax.experimental import pallas as pl
from jax.experimental.pallas import tpu as pltpu
from jax.experimental.pallas import tpu_sc as plsc

N, D, S, H, O = 10000, 128, 32, 128, 128

NC = 2
NS = 16
NPAD = 10240
CB = 8
CS = CB * S
RB0 = 640
RB1 = 0
assert RB0 * NS + RB1 * NS == NPAD
IDXW = RB0 * S
IDXPAD = (NPAD + RB0) * S


@functools.cache
def _make_gather_sum():
    mesh = plsc.VectorSubcoreMesh(core_axis_name="c", subcore_axis_name="s",
                                  num_cores=1)

    @functools.partial(
        pl.kernel,
        mesh=mesh,
        out_type=jax.ShapeDtypeStruct((NPAD, D), jnp.float32),
        scratch_types=[
            pltpu.VMEM((IDXW,), jnp.int32),
            pltpu.VMEM((2, CS, D), jnp.float32),
            pltpu.VMEM((2, CB, D), jnp.float32),
            pltpu.SemaphoreType.DMA,
            pltpu.SemaphoreType.DMA,
            pltpu.SemaphoreType.DMA,
            pltpu.SemaphoreType.DMA,
        ],
    )
    def _gather_sum(table, idxf, out, idx_v, buf_v, acc_v, g0, g1, o0, o1):
        gsem = (g0, g1)
        osem = (o0, o1)
        sid = lax.axis_index("s")
        cid = lax.axis_index("c")
        base = jnp.where(cid == 0, sid * RB0, NS * RB0 + sid * RB1)
        npairs = jnp.where(cid == 0, RB0 // (2 * CB), RB1 // (2 * CB))
        nchunk = 2 * npairs

        @pl.when(npairs > 0)
        def _worker():
            pltpu.sync_copy(idxf.at[pl.ds(base * S, IDXW)], idx_v)

            def issue_gather(c, b):
                pltpu.async_copy(table.at[idx_v.at[pl.ds(c * CS, CS)]],
                                 buf_v.at[b], gsem[b])

            issue_gather(0, 0)
            issue_gather(1, 1)

            def pair_body(i, carry):
                for b in range(2):
                    c = 2 * i + b
                    pltpu.make_async_copy(table.at[idx_v.at[pl.ds(0, CS)]],
                                          buf_v.at[b], gsem[b]).wait()

                    @pl.when(i > 0)
                    def _():
                        pltpu.make_async_copy(acc_v.at[b],
                                              out.at[pl.ds(0, CB)],
                                              osem[b]).wait()

                    def red_body(r, cr):
                        for col in range(D // 16):
                            csl = pl.ds(col * 16, 16)
                            a = [buf_v[b, r * S + k, csl] for k in range(4)]
                            for s in range(4, S, 4):
                                for k in range(4):
                                    a[k] = a[k] + buf_v[b, r * S + s + k, csl]
                            acc_v[b, r, csl] = (a[0] + a[1]) + (a[2] + a[3])
                        return cr

                    lax.fori_loop(0, CB, red_body, 0, unroll=False)

                    pltpu.async_copy(acc_v.at[b],
                                     out.at[pl.ds(base + c * CB, CB)],
                                     osem[b])

                    @pl.when(c + 2 < nchunk)
                    def _():
                        issue_gather(c + 2, b)
                return carry

            lax.fori_loop(0, npairs, pair_body, 0, unroll=False)
            for b in range(2):
                pltpu.make_async_copy(acc_v.at[b], out.at[pl.ds(0, CB)],
                                      osem[b]).wait()

    return _gather_sum


BLK = 1000


def _layer1_body(f_ref, s1_ref, w1t_ref, w1b_ref, b1_ref, p_ref, st_ref):
    x = jnp.dot(f_ref[...], w1t_ref[...], preferred_element_type=jnp.float32)
    x = x + jnp.dot(s1_ref[...] * (1.0 / S), w1b_ref[...],
                    preferred_element_type=jnp.float32)
    x = x + b1_ref[...]
    p = jnp.maximum(x, 0.0)
    p_ref[...] = p
    ssum = jnp.sum(p, axis=0, keepdims=True)
    ssq = jnp.sum(p * p, axis=0, keepdims=True)

    @pl.when(pl.program_id(0) == 0)
    def _():
        st_ref[...] = jnp.zeros_like(st_ref)

    st_ref[0:1, :] += ssum
    st_ref[1:2, :] += ssq


def _layer2_body(p_ref, s2_ref, w2t_ref, w2b_ref, st_ref, g_ref, be_ref,
                 b2_ref, o_ref):
    mu = st_ref[0:1, :] * (1.0 / N)
    ex2 = st_ref[1:2, :] * (1.0 / N)
    var = ex2 - mu * mu
    a = g_ref[...] * lax.rsqrt(var + 1e-5)
    c = be_ref[...] - mu * a
    ph = p_ref[...] * a + c
    ag = (s2_ref[...] * (1.0 / S)) * a + c
    o_ref[...] = (jnp.dot(ph, w2t_ref[...], preferred_element_type=jnp.float32)
                  + jnp.dot(ag, w2b_ref[...], preferred_element_type=jnp.float32)
                  + b2_ref[...])


def kernel(features, neigh_idx, W1, b1, gamma, beta, W2, b2):
    idx_flat = neigh_idx.reshape(-1)
    idx_flat = jnp.concatenate(
        [idx_flat, jnp.zeros((IDXPAD - N * S,), dtype=idx_flat.dtype)])

    gather_sum = _make_gather_sum()
    s1 = gather_sum(features, idx_flat)

    w1t, w1b = W1[:D], W1[D:]
    p, stats = pl.pallas_call(
        _layer1_body,
        grid=(N // BLK,),
        in_specs=[
            pl.BlockSpec((BLK, D), lambda i: (i, 0)),
            pl.BlockSpec((BLK, D), lambda i: (i, 0)),
            pl.BlockSpec((D, H), lambda i: (0, 0)),
            pl.BlockSpec((D, H), lambda i: (0, 0)),
            pl.BlockSpec((1, H), lambda i: (0, 0)),
        ],
        out_specs=[
            pl.BlockSpec((BLK, H), lambda i: (i, 0)),
            pl.BlockSpec((8, H), lambda i: (0, 0)),
        ],
        out_shape=[
            jax.ShapeDtypeStruct((N, H), jnp.float32),
            jax.ShapeDtypeStruct((8, H), jnp.float32),
        ],
    )(features, s1, w1t, w1b, b1.reshape(1, H))

    s2 = gather_sum(p, idx_flat)

    w2t, w2b = W2[:H], W2[H:]
    out = pl.pallas_call(
        _layer2_body,
        grid=(N // BLK,),
        in_specs=[
            pl.BlockSpec((BLK, H), lambda i: (i, 0)),
            pl.BlockSpec((BLK, H), lambda i: (i, 0)),
            pl.BlockSpec((H, O), lambda i: (0, 0)),
            pl.BlockSpec((H, O), lambda i: (0, 0)),
            pl.BlockSpec((8, H), lambda i: (0, 0)),
            pl.BlockSpec((1, H), lambda i: (0, 0)),
            pl.BlockSpec((1, H), lambda i: (0, 0)),
            pl.BlockSpec((1, O), lambda i: (0, 0)),
        ],
        out_specs=pl.BlockSpec((BLK, O), lambda i: (i, 0)),
        out_shape=jax.ShapeDtypeStruct((N, O), jnp.float32),
    )(p, s2, w2t, w2b, stats, gamma.reshape(1, H), beta.reshape(1, H),
      b2.reshape(1, O))
    return out


# speedup vs baseline: 1.0020x; 1.0020x over previous
import functools

import jax
import jax.numpy as jnp
from jax import lax
from jax.experimental import pallas as pl
from jax.experimental.pallas import tpu as pltpu
from jax.experimental.pallas import tpu_sc as plsc

N, D, S, H, O = 10000, 128, 32, 128, 128

NC = 2
NS = 16
NPAD = 10240
CB = 8
CS = CB * S
RB0 = NPAD // NS
NCHUNK = RB0 // CB
NPAIR = NCHUNK // 2
IDXW = RB0 * S
IDXPAD = NPAD * S


@functools.cache
def _make_gather_sum():
    mesh = plsc.VectorSubcoreMesh(core_axis_name="c", subcore_axis_name="s",
                                  num_cores=1)

    @functools.partial(
        pl.kernel,
        mesh=mesh,
        out_type=jax.ShapeDtypeStruct((NPAD, D), jnp.float32),
        scratch_types=[
            pltpu.VMEM((IDXW,), jnp.int32),
            pltpu.VMEM((2, CS, D), jnp.float32),
            pltpu.VMEM((2, CB, D), jnp.float32),
            pltpu.SemaphoreType.DMA,
            pltpu.SemaphoreType.DMA,
            pltpu.SemaphoreType.DMA,
            pltpu.SemaphoreType.DMA,
        ],
    )
    def _gather_sum(table, idxf, out, idx_v, buf_v, acc_v, g0, g1, o0, o1):
        gsem = (g0, g1)
        osem = (o0, o1)
        sid = lax.axis_index("s")
        base = sid * RB0
        pltpu.sync_copy(idxf.at[pl.ds(base * S, IDXW)], idx_v)

        def issue_gather(c, b):
            pltpu.async_copy(table.at[idx_v.at[pl.ds(c * CS, CS)]],
                             buf_v.at[b], gsem[b])

        issue_gather(0, 0)
        issue_gather(1, 1)

        def pair_body(i, carry):
            for b in range(2):
                c = 2 * i + b
                pltpu.make_async_copy(table.at[idx_v.at[pl.ds(0, CS)]],
                                      buf_v.at[b], gsem[b]).wait()

                @pl.when(i > 0)
                def _():
                    pltpu.make_async_copy(acc_v.at[b], out.at[pl.ds(0, CB)],
                                          osem[b]).wait()

                def red_body(r, cr):
                    for col in range(D // 16):
                        csl = pl.ds(col * 16, 16)
                        a = [buf_v[b, r * S + k, csl] for k in range(4)]
                        for s in range(4, S, 4):
                            for k in range(4):
                                a[k] = a[k] + buf_v[b, r * S + s + k, csl]
                        acc_v[b, r, csl] = (a[0] + a[1]) + (a[2] + a[3])
                    return cr

                lax.fori_loop(0, CB, red_body, 0, unroll=False)

                pltpu.async_copy(acc_v.at[b],
                                 out.at[pl.ds(base + c * CB, CB)], osem[b])

                @pl.when(c + 2 < NCHUNK)
                def _():
                    issue_gather(c + 2, b)
            return carry

        lax.fori_loop(0, NPAIR, pair_body, 0, unroll=False)
        for b in range(2):
            pltpu.make_async_copy(acc_v.at[b], out.at[pl.ds(0, CB)],
                                  osem[b]).wait()

    return _gather_sum


BLK = 1000


def _layer1_body(f_ref, s1_ref, w1t_ref, w1b_ref, b1_ref, p_ref, st_ref):
    x = jnp.dot(f_ref[...], w1t_ref[...], preferred_element_type=jnp.float32)
    x = x + jnp.dot(s1_ref[...] * (1.0 / S), w1b_ref[...],
                    preferred_element_type=jnp.float32)
    x = x + b1_ref[...]
    p = jnp.maximum(x, 0.0)
    p_ref[...] = p
    ssum = jnp.sum(p, axis=0, keepdims=True)
    ssq = jnp.sum(p * p, axis=0, keepdims=True)

    @pl.when(pl.program_id(0) == 0)
    def _():
        st_ref[...] = jnp.zeros_like(st_ref)

    st_ref[0:1, :] += ssum
    st_ref[1:2, :] += ssq


def _layer2_body(p_ref, s2_ref, w2t_ref, w2b_ref, st_ref, g_ref, be_ref,
                 b2_ref, o_ref):
    mu = st_ref[0:1, :] * (1.0 / N)
    ex2 = st_ref[1:2, :] * (1.0 / N)
    var = ex2 - mu * mu
    a = g_ref[...] * lax.rsqrt(var + 1e-5)
    c = be_ref[...] - mu * a
    ph = p_ref[...] * a + c
    ag = (s2_ref[...] * (1.0 / S)) * a + c
    o_ref[...] = (jnp.dot(ph, w2t_ref[...], preferred_element_type=jnp.float32)
                  + jnp.dot(ag, w2b_ref[...], preferred_element_type=jnp.float32)
                  + b2_ref[...])


def kernel(features, neigh_idx, W1, b1, gamma, beta, W2, b2):
    idx_flat = neigh_idx.reshape(-1)
    idx_flat = jnp.concatenate(
        [idx_flat, jnp.zeros((IDXPAD - N * S,), dtype=idx_flat.dtype)])

    gather_sum = _make_gather_sum()
    s1 = gather_sum(features, idx_flat)

    w1t, w1b = W1[:D], W1[D:]
    p, stats = pl.pallas_call(
        _layer1_body,
        grid=(N // BLK,),
        in_specs=[
            pl.BlockSpec((BLK, D), lambda i: (i, 0)),
            pl.BlockSpec((BLK, D), lambda i: (i, 0)),
            pl.BlockSpec((D, H), lambda i: (0, 0)),
            pl.BlockSpec((D, H), lambda i: (0, 0)),
            pl.BlockSpec((1, H), lambda i: (0, 0)),
        ],
        out_specs=[
            pl.BlockSpec((BLK, H), lambda i: (i, 0)),
            pl.BlockSpec((8, H), lambda i: (0, 0)),
        ],
        out_shape=[
            jax.ShapeDtypeStruct((N, H), jnp.float32),
            jax.ShapeDtypeStruct((8, H), jnp.float32),
        ],
    )(features, s1, w1t, w1b, b1.reshape(1, H))

    s2 = gather_sum(p, idx_flat)

    w2t, w2b = W2[:H], W2[H:]
    out = pl.pallas_call(
        _layer2_body,
        grid=(N // BLK,),
        in_specs=[
            pl.BlockSpec((BLK, H), lambda i: (i, 0)),
            pl.BlockSpec((BLK, H), lambda i: (i, 0)),
            pl.BlockSpec((H, O), lambda i: (0, 0)),
            pl.BlockSpec((H, O), lambda i: (0, 0)),
            pl.BlockSpec((8, H), lambda i: (0, 0)),
            pl.BlockSpec((1, H), lambda i: (0, 0)),
            pl.BlockSpec((1, H), lambda i: (0, 0)),
            pl.BlockSpec((1, O), lambda i: (0, 0)),
        ],
        out_specs=pl.BlockSpec((BLK, O), lambda i: (i, 0)),
        out_shape=jax.ShapeDtypeStruct((N, O), jnp.float32),
    )(p, s2, w2t, w2b, stats, gamma.reshape(1, H), beta.reshape(1, H),
      b2.reshape(1, O))
    return out


# speedup vs baseline: 1.0132x; 1.0112x over previous
import functools

import jax
import jax.numpy as jnp
from jax import lax
from jax.experimental import pallas as pl
from jax.experimental.pallas import tpu as pltpu
from jax.experimental.pallas import tpu_sc as plsc

N, D, S, H, O = 10000, 128, 32, 128, 128

NC = 2
NS = 16
NPAD = 10240
CB = 8
CS = CB * S
NSLOT = 3
RB0 = NPAD // NS
NCHUNK = RB0 // CB
NGRP = NCHUNK // NSLOT
NREM = NCHUNK - NGRP * NSLOT
IDXW = RB0 * S
IDXPAD = NPAD * S


@functools.cache
def _make_gather_sum():
    mesh = plsc.VectorSubcoreMesh(core_axis_name="c", subcore_axis_name="s",
                                  num_cores=1)

    @functools.partial(
        pl.kernel,
        mesh=mesh,
        out_type=jax.ShapeDtypeStruct((NPAD, D), jnp.float32),
        scratch_types=[
            pltpu.VMEM((IDXW,), jnp.int32),
            pltpu.VMEM((NSLOT, CS, D), jnp.float32),
            pltpu.VMEM((NSLOT, CB, D), jnp.float32),
            pltpu.SemaphoreType.DMA,
            pltpu.SemaphoreType.DMA,
            pltpu.SemaphoreType.DMA,
            pltpu.SemaphoreType.DMA,
            pltpu.SemaphoreType.DMA,
            pltpu.SemaphoreType.DMA,
        ],
    )
    def _gather_sum(table, idxf, out, idx_v, buf_v, acc_v,
                    g0, g1, g2, o0, o1, o2):
        gsem = (g0, g1, g2)
        osem = (o0, o1, o2)
        sid = lax.axis_index("s")
        base = sid * RB0
        pltpu.sync_copy(idxf.at[pl.ds(base * S, IDXW)], idx_v)

        def issue_gather(c, b):
            pltpu.async_copy(table.at[idx_v.at[pl.ds(c * CS, CS)]],
                             buf_v.at[b], gsem[b])

        for b in range(NSLOT):
            issue_gather(b, b)

        def do_chunk(c, b, first, last_issue):
            pltpu.make_async_copy(table.at[idx_v.at[pl.ds(0, CS)]],
                                  buf_v.at[b], gsem[b]).wait()

            if first is None:
                pltpu.make_async_copy(acc_v.at[b], out.at[pl.ds(0, CB)],
                                      osem[b]).wait()
            else:
                @pl.when(jnp.logical_not(first))
                def _():
                    pltpu.make_async_copy(acc_v.at[b], out.at[pl.ds(0, CB)],
                                          osem[b]).wait()

            def red_body(r, cr):
                for col in range(D // 16):
                    csl = pl.ds(col * 16, 16)
                    a = [buf_v[b, r * S + k, csl] for k in range(4)]
                    for s in range(4, S, 4):
                        for k in range(4):
                            a[k] = a[k] + buf_v[b, r * S + s + k, csl]
                    acc_v[b, r, csl] = (a[0] + a[1]) + (a[2] + a[3])
                return cr

            lax.fori_loop(0, CB, red_body, 0, unroll=False)

            pltpu.async_copy(acc_v.at[b],
                             out.at[pl.ds(base + c * CB, CB)], osem[b])

            if last_issue:
                @pl.when(c + NSLOT < NCHUNK)
                def _():
                    issue_gather(c + NSLOT, b)

        def grp_body(i, carry):
            for b in range(NSLOT):
                do_chunk(i * NSLOT + b, b, i == 0, True)
            return carry

        lax.fori_loop(0, NGRP, grp_body, 0, unroll=False)
        for r in range(NREM):
            do_chunk(NGRP * NSLOT + r, r, None, False)
        for b in range(NSLOT):
            pltpu.make_async_copy(acc_v.at[b], out.at[pl.ds(0, CB)],
                                  osem[b]).wait()

    return _gather_sum


BLK = 1000


def _layer1_body(f_ref, s1_ref, w1t_ref, w1b_ref, b1_ref, p_ref, st_ref):
    x = jnp.dot(f_ref[...], w1t_ref[...], preferred_element_type=jnp.float32)
    x = x + jnp.dot(s1_ref[...] * (1.0 / S), w1b_ref[...],
                    preferred_element_type=jnp.float32)
    x = x + b1_ref[...]
    p = jnp.maximum(x, 0.0)
    p_ref[...] = p
    ssum = jnp.sum(p, axis=0, keepdims=True)
    ssq = jnp.sum(p * p, axis=0, keepdims=True)

    @pl.when(pl.program_id(0) == 0)
    def _():
        st_ref[...] = jnp.zeros_like(st_ref)

    st_ref[0:1, :] += ssum
    st_ref[1:2, :] += ssq


def _layer2_body(p_ref, s2_ref, w2t_ref, w2b_ref, st_ref, g_ref, be_ref,
                 b2_ref, o_ref):
    mu = st_ref[0:1, :] * (1.0 / N)
    ex2 = st_ref[1:2, :] * (1.0 / N)
    var = ex2 - mu * mu
    a = g_ref[...] * lax.rsqrt(var + 1e-5)
    c = be_ref[...] - mu * a
    ph = p_ref[...] * a + c
    ag = (s2_ref[...] * (1.0 / S)) * a + c
    o_ref[...] = (jnp.dot(ph, w2t_ref[...], preferred_element_type=jnp.float32)
                  + jnp.dot(ag, w2b_ref[...], preferred_element_type=jnp.float32)
                  + b2_ref[...])


def kernel(features, neigh_idx, W1, b1, gamma, beta, W2, b2):
    idx_flat = neigh_idx.reshape(-1)
    idx_flat = jnp.concatenate(
        [idx_flat, jnp.zeros((IDXPAD - N * S,), dtype=idx_flat.dtype)])

    gather_sum = _make_gather_sum()
    s1 = gather_sum(features, idx_flat)

    w1t, w1b = W1[:D], W1[D:]
    p, stats = pl.pallas_call(
        _layer1_body,
        grid=(N // BLK,),
        in_specs=[
            pl.BlockSpec((BLK, D), lambda i: (i, 0)),
            pl.BlockSpec((BLK, D), lambda i: (i, 0)),
            pl.BlockSpec((D, H), lambda i: (0, 0)),
            pl.BlockSpec((D, H), lambda i: (0, 0)),
            pl.BlockSpec((1, H), lambda i: (0, 0)),
        ],
        out_specs=[
            pl.BlockSpec((BLK, H), lambda i: (i, 0)),
            pl.BlockSpec((8, H), lambda i: (0, 0)),
        ],
        out_shape=[
            jax.ShapeDtypeStruct((N, H), jnp.float32),
            jax.ShapeDtypeStruct((8, H), jnp.float32),
        ],
    )(features, s1, w1t, w1b, b1.reshape(1, H))

    s2 = gather_sum(p, idx_flat)

    w2t, w2b = W2[:H], W2[H:]
    out = pl.pallas_call(
        _layer2_body,
        grid=(N // BLK,),
        in_specs=[
            pl.BlockSpec((BLK, H), lambda i: (i, 0)),
            pl.BlockSpec((BLK, H), lambda i: (i, 0)),
            pl.BlockSpec((H, O), lambda i: (0, 0)),
            pl.BlockSpec((H, O), lambda i: (0, 0)),
            pl.BlockSpec((8, H), lambda i: (0, 0)),
            pl.BlockSpec((1, H), lambda i: (0, 0)),
            pl.BlockSpec((1, H), lambda i: (0, 0)),
            pl.BlockSpec((1, O), lambda i: (0, 0)),
        ],
        out_specs=pl.BlockSpec((BLK, O), lambda i: (i, 0)),
        out_shape=jax.ShapeDtypeStruct((N, O), jnp.float32),
    )(p, s2, w2t, w2b, stats, gamma.reshape(1, H), beta.reshape(1, H),
      b2.reshape(1, O))
    return out


# speedup vs baseline: 1.1895x; 1.1740x over previous
import functools

import jax
import jax.numpy as jnp
from jax import lax
from jax.experimental import pallas as pl
from jax.experimental.pallas import tpu as pltpu
from jax.experimental.pallas import tpu_sc as plsc

N, D, S, H, O = 10000, 128, 32, 128, 128

NC = 2
NS = 16
NPAD = 10240
CB = 8
CS = CB * S
RB0 = 624
RB1 = 16
assert RB0 * NS + RB1 * NS == NPAD
IDXW = RB0 * S
IDXPAD = (NPAD + RB0) * S


@functools.cache
def _make_gather_sum():
    mesh = plsc.VectorSubcoreMesh(core_axis_name="c", subcore_axis_name="s")

    @functools.partial(
        pl.kernel,
        mesh=mesh,
        out_type=jax.ShapeDtypeStruct((NPAD, D), jnp.float32),
        scratch_types=[
            pltpu.VMEM((IDXW,), jnp.int32),
            pltpu.VMEM((2, CS, D), jnp.float32),
            pltpu.VMEM((2, CB, D), jnp.float32),
            pltpu.SemaphoreType.DMA,
            pltpu.SemaphoreType.DMA,
            pltpu.SemaphoreType.DMA,
            pltpu.SemaphoreType.DMA,
        ],
    )
    def _gather_sum(table, idxf, out, idx_v, buf_v, acc_v, g0, g1, o0, o1):
        gsem = (g0, g1)
        osem = (o0, o1)
        sid = lax.axis_index("s")
        cid = lax.axis_index("c")
        base = jnp.where(cid == 0, sid * RB0, NS * RB0 + sid * RB1)
        npairs = jnp.where(cid == 0, RB0 // (2 * CB), RB1 // (2 * CB))
        nchunk = 2 * npairs
        pltpu.sync_copy(idxf.at[pl.ds(base * S, IDXW)], idx_v)

        def issue_gather(c, b):
            pltpu.async_copy(table.at[idx_v.at[pl.ds(c * CS, CS)]],
                             buf_v.at[b], gsem[b])

        issue_gather(0, 0)
        issue_gather(1, 1)

        def pair_body(i, carry):
            for b in range(2):
                c = 2 * i + b
                pltpu.make_async_copy(table.at[idx_v.at[pl.ds(0, CS)]],
                                      buf_v.at[b], gsem[b]).wait()

                @pl.when(i > 0)
                def _():
                    pltpu.make_async_copy(acc_v.at[b], out.at[pl.ds(0, CB)],
                                          osem[b]).wait()

                def red_body(r, cr):
                    for col in range(D // 16):
                        csl = pl.ds(col * 16, 16)
                        a = [buf_v[b, r * S + k, csl] for k in range(4)]
                        for s in range(4, S, 4):
                            for k in range(4):
                                a[k] = a[k] + buf_v[b, r * S + s + k, csl]
                        acc_v[b, r, csl] = (a[0] + a[1]) + (a[2] + a[3])
                    return cr

                lax.fori_loop(0, CB, red_body, 0, unroll=False)

                pltpu.async_copy(acc_v.at[b],
                                 out.at[pl.ds(base + c * CB, CB)], osem[b])

                @pl.when(c + 2 < nchunk)
                def _():
                    issue_gather(c + 2, b)
            return carry

        lax.fori_loop(0, npairs, pair_body, 0, unroll=False)
        for b in range(2):
            pltpu.make_async_copy(acc_v.at[b], out.at[pl.ds(0, CB)],
                                  osem[b]).wait()

    return _gather_sum


BLK = 1000


def _layer1_body(f_ref, s1_ref, w1t_ref, w1b_ref, b1_ref, p_ref, st_ref):
    x = jnp.dot(f_ref[...], w1t_ref[...], preferred_element_type=jnp.float32)
    x = x + jnp.dot(s1_ref[...] * (1.0 / S), w1b_ref[...],
                    preferred_element_type=jnp.float32)
    x = x + b1_ref[...]
    p = jnp.maximum(x, 0.0)
    p_ref[...] = p
    ssum = jnp.sum(p, axis=0, keepdims=True)
    ssq = jnp.sum(p * p, axis=0, keepdims=True)

    @pl.when(pl.program_id(0) == 0)
    def _():
        st_ref[...] = jnp.zeros_like(st_ref)

    st_ref[0:1, :] += ssum
    st_ref[1:2, :] += ssq


def _layer2_body(p_ref, s2_ref, w2t_ref, w2b_ref, st_ref, g_ref, be_ref,
                 b2_ref, o_ref):
    mu = st_ref[0:1, :] * (1.0 / N)
    ex2 = st_ref[1:2, :] * (1.0 / N)
    var = ex2 - mu * mu
    a = g_ref[...] * lax.rsqrt(var + 1e-5)
    c = be_ref[...] - mu * a
    ph = p_ref[...] * a + c
    ag = (s2_ref[...] * (1.0 / S)) * a + c
    o_ref[...] = (jnp.dot(ph, w2t_ref[...], preferred_element_type=jnp.float32)
                  + jnp.dot(ag, w2b_ref[...], preferred_element_type=jnp.float32)
                  + b2_ref[...])


def kernel(features, neigh_idx, W1, b1, gamma, beta, W2, b2):
    idx_flat = neigh_idx.reshape(-1)
    idx_flat = jnp.concatenate(
        [idx_flat, jnp.zeros((IDXPAD - N * S,), dtype=idx_flat.dtype)])

    gather_sum = _make_gather_sum()
    s1 = gather_sum(features, idx_flat)

    w1t, w1b = W1[:D], W1[D:]
    p, stats = pl.pallas_call(
        _layer1_body,
        grid=(N // BLK,),
        in_specs=[
            pl.BlockSpec((BLK, D), lambda i: (i, 0)),
            pl.BlockSpec((BLK, D), lambda i: (i, 0)),
            pl.BlockSpec((D, H), lambda i: (0, 0)),
            pl.BlockSpec((D, H), lambda i: (0, 0)),
            pl.BlockSpec((1, H), lambda i: (0, 0)),
        ],
        out_specs=[
            pl.BlockSpec((BLK, H), lambda i: (i, 0)),
            pl.BlockSpec((8, H), lambda i: (0, 0)),
        ],
        out_shape=[
            jax.ShapeDtypeStruct((N, H), jnp.float32),
            jax.ShapeDtypeStruct((8, H), jnp.float32),
        ],
    )(features, s1, w1t, w1b, b1.reshape(1, H))

    s2 = gather_sum(p, idx_flat)

    w2t, w2b = W2[:H], W2[H:]
    out = pl.pallas_call(
        _layer2_body,
        grid=(N // BLK,),
        in_specs=[
            pl.BlockSpec((BLK, H), lambda i: (i, 0)),
            pl.BlockSpec((BLK, H), lambda i: (i, 0)),
            pl.BlockSpec((H, O), lambda i: (0, 0)),
            pl.BlockSpec((H, O), lambda i: (0, 0)),
            pl.BlockSpec((8, H), lambda i: (0, 0)),
            pl.BlockSpec((1, H), lambda i: (0, 0)),
            pl.BlockSpec((1, H), lambda i: (0, 0)),
            pl.BlockSpec((1, O), lambda i: (0, 0)),
        ],
        out_specs=pl.BlockSpec((BLK, O), lambda i: (i, 0)),
        out_shape=jax.ShapeDtypeStruct((N, O), jnp.float32),
    )(p, s2, w2t, w2b, stats, gamma.reshape(1, H), beta.reshape(1, H),
      b2.reshape(1, O))
    return out


# speedup vs baseline: 1.1918x; 1.0019x over previous
import functools

import jax
import jax.numpy as jnp
from jax import lax
from jax.experimental import pallas as pl
from jax.experimental.pallas import tpu as pltpu
from jax.experimental.pallas import tpu_sc as plsc

N, D, S, H, O = 10000, 128, 32, 128, 128

NC = 2
NS = 16
NPAD = 10240
CB = 8
CS = CB * S
RB0 = 624
RB1 = 16
assert RB0 * NS + RB1 * NS == NPAD
IDXW = RB0 * S
IDXPAD = NPAD * S


@functools.cache
def _make_gather_sum():
    mesh = plsc.VectorSubcoreMesh(core_axis_name="c", subcore_axis_name="s")

    @functools.partial(
        pl.kernel,
        mesh=mesh,
        out_type=jax.ShapeDtypeStruct((NPAD, D), jnp.float32),
        scratch_types=[
            pltpu.VMEM((IDXW,), jnp.int32),
            pltpu.VMEM((2, CS, D), jnp.float32),
            pltpu.VMEM((2, CB, D), jnp.float32),
            pltpu.SemaphoreType.DMA,
            pltpu.SemaphoreType.DMA,
            pltpu.SemaphoreType.DMA,
            pltpu.SemaphoreType.DMA,
        ],
    )
    def _gather_sum(table, idxf, out, idx_v, buf_v, acc_v, g0, g1, o0, o1):
        gsem = (g0, g1)
        osem = (o0, o1)
        sid = lax.axis_index("s")
        cid = lax.axis_index("c")
        base = jnp.where(cid == 0, sid * RB0, NS * RB0 + sid * RB1)
        npairs = jnp.where(cid == 0, RB0 // (2 * CB), RB1 // (2 * CB))
        nchunk = 2 * npairs
        @pl.when(cid == 0)
        def _():
            pltpu.sync_copy(idxf.at[pl.ds(base * S, IDXW)], idx_v)

        @pl.when(cid == 1)
        def _():
            pltpu.sync_copy(idxf.at[pl.ds(base * S, RB1 * S)],
                            idx_v.at[pl.ds(0, RB1 * S)])

        def issue_gather(c, b):
            pltpu.async_copy(table.at[idx_v.at[pl.ds(c * CS, CS)]],
                             buf_v.at[b], gsem[b])

        issue_gather(0, 0)
        issue_gather(1, 1)

        def pair_body(i, carry):
            for b in range(2):
                c = 2 * i + b
                pltpu.make_async_copy(table.at[idx_v.at[pl.ds(0, CS)]],
                                      buf_v.at[b], gsem[b]).wait()

                @pl.when(i > 0)
                def _():
                    pltpu.make_async_copy(acc_v.at[b], out.at[pl.ds(0, CB)],
                                          osem[b]).wait()

                def red_body(r, cr):
                    for col in range(D // 16):
                        csl = pl.ds(col * 16, 16)
                        a = [buf_v[b, r * S + k, csl] for k in range(4)]
                        for s in range(4, S, 4):
                            for k in range(4):
                                a[k] = a[k] + buf_v[b, r * S + s + k, csl]
                        acc_v[b, r, csl] = (a[0] + a[1]) + (a[2] + a[3])
                    return cr

                lax.fori_loop(0, CB, red_body, 0, unroll=False)

                pltpu.async_copy(acc_v.at[b],
                                 out.at[pl.ds(base + c * CB, CB)], osem[b])

                @pl.when(c + 2 < nchunk)
                def _():
                    issue_gather(c + 2, b)
            return carry

        lax.fori_loop(0, npairs, pair_body, 0, unroll=False)
        for b in range(2):
            pltpu.make_async_copy(acc_v.at[b], out.at[pl.ds(0, CB)],
                                  osem[b]).wait()

    return _gather_sum


BLK = 1000


def _layer1_body(f_ref, s1_ref, w1t_ref, w1b_ref, b1_ref, p_ref, st_ref):
    x = jnp.dot(f_ref[...], w1t_ref[...], preferred_element_type=jnp.float32)
    x = x + jnp.dot(s1_ref[...] * (1.0 / S), w1b_ref[...],
                    preferred_element_type=jnp.float32)
    x = x + b1_ref[...]
    p = jnp.maximum(x, 0.0)
    p_ref[...] = p
    ssum = jnp.sum(p, axis=0, keepdims=True)
    ssq = jnp.sum(p * p, axis=0, keepdims=True)

    @pl.when(pl.program_id(0) == 0)
    def _():
        st_ref[...] = jnp.zeros_like(st_ref)

    st_ref[0:1, :] += ssum
    st_ref[1:2, :] += ssq


def _layer2_body(p_ref, s2_ref, w2t_ref, w2b_ref, st_ref, g_ref, be_ref,
                 b2_ref, o_ref):
    mu = st_ref[0:1, :] * (1.0 / N)
    ex2 = st_ref[1:2, :] * (1.0 / N)
    var = ex2 - mu * mu
    a = g_ref[...] * lax.rsqrt(var + 1e-5)
    c = be_ref[...] - mu * a
    ph = p_ref[...] * a + c
    ag = (s2_ref[...] * (1.0 / S)) * a + c
    o_ref[...] = (jnp.dot(ph, w2t_ref[...], preferred_element_type=jnp.float32)
                  + jnp.dot(ag, w2b_ref[...], preferred_element_type=jnp.float32)
                  + b2_ref[...])


def kernel(features, neigh_idx, W1, b1, gamma, beta, W2, b2):
    idx_flat = neigh_idx.reshape(-1)
    idx_flat = jnp.concatenate(
        [idx_flat, jnp.zeros((IDXPAD - N * S,), dtype=idx_flat.dtype)])

    gather_sum = _make_gather_sum()
    s1 = gather_sum(features, idx_flat)

    w1t, w1b = W1[:D], W1[D:]
    p, stats = pl.pallas_call(
        _layer1_body,
        grid=(N // BLK,),
        in_specs=[
            pl.BlockSpec((BLK, D), lambda i: (i, 0)),
            pl.BlockSpec((BLK, D), lambda i: (i, 0)),
            pl.BlockSpec((D, H), lambda i: (0, 0)),
            pl.BlockSpec((D, H), lambda i: (0, 0)),
            pl.BlockSpec((1, H), lambda i: (0, 0)),
        ],
        out_specs=[
            pl.BlockSpec((BLK, H), lambda i: (i, 0)),
            pl.BlockSpec((8, H), lambda i: (0, 0)),
        ],
        out_shape=[
            jax.ShapeDtypeStruct((N, H), jnp.float32),
            jax.ShapeDtypeStruct((8, H), jnp.float32),
        ],
    )(features, s1, w1t, w1b, b1.reshape(1, H))

    s2 = gather_sum(p, idx_flat)

    w2t, w2b = W2[:H], W2[H:]
    out = pl.pallas_call(
        _layer2_body,
        grid=(N // BLK,),
        in_specs=[
            pl.BlockSpec((BLK, H), lambda i: (i, 0)),
            pl.BlockSpec((BLK, H), lambda i: (i, 0)),
            pl.BlockSpec((H, O), lambda i: (0, 0)),
            pl.BlockSpec((H, O), lambda i: (0, 0)),
            pl.BlockSpec((8, H), lambda i: (0, 0)),
            pl.BlockSpec((1, H), lambda i: (0, 0)),
            pl.BlockSpec((1, H), lambda i: (0, 0)),
            pl.BlockSpec((1, O), lambda i: (0, 0)),
        ],
        out_specs=pl.BlockSpec((BLK, O), lambda i: (i, 0)),
        out_shape=jax.ShapeDtypeStruct((N, O), jnp.float32),
    )(p, s2, w2t, w2b, stats, gamma.reshape(1, H), beta.reshape(1, H),
      b2.reshape(1, O))
    return out
